# baseline (speedup 1.0000x reference)
import jax
import jax.numpy as jnp
from jax import lax
from jax.experimental import pallas as pl
from jax.experimental.pallas import tpu as pltpu
from jax.experimental.pallas import tpu_sc as plsc

N_NODES = 10000
N_EDGES = 320000
D = 128
L = 16
NCH = D // L

NC = 2
NS = 16
NW = NC * NS
EB = 64
NBLK = N_EDGES // EB
BPT = 156
SUP = 3
PAIR = 2 * SUP
NPAIRS = BPT // PAIR
TILE_EDGES = BPT * EB
LEFTOVER = NBLK - BPT * NW
LEFT_BASE = BPT * NW * EB
ROWS_PER_TILE = 624
TAIL_ROWS = N_NODES - NS * ROWS_PER_TILE

_SC_MESH = dict(core_axis_name="c", subcore_axis_name="s",
                num_cores=NC, num_subcores=NS)


def _proj_body(s_ref, r_ref, ws_ref, bs_ref, wr_ref, br_ref, ps_ref, pr_ref):
    ps_ref[...] = jnp.dot(s_ref[...], ws_ref[...],
                          preferred_element_type=jnp.float32) + bs_ref[...]
    pr_ref[...] = jnp.dot(r_ref[...], wr_ref[...],
                          preferred_element_type=jnp.float32) + br_ref[...]


def _project(s_embed, r_embed, W_s, b_s, W_r, b_r):
    RB = 2000
    grid = (N_NODES // RB,)
    return pl.pallas_call(
        _proj_body,
        grid=grid,
        in_specs=[
            pl.BlockSpec((RB, D), lambda i: (i, 0)),
            pl.BlockSpec((RB, D), lambda i: (i, 0)),
            pl.BlockSpec((D, D), lambda i: (0, 0)),
            pl.BlockSpec((1, D), lambda i: (0, 0)),
            pl.BlockSpec((D, D), lambda i: (0, 0)),
            pl.BlockSpec((1, D), lambda i: (0, 0)),
        ],
        out_specs=[
            pl.BlockSpec((RB, D), lambda i: (i, 0)),
            pl.BlockSpec((RB, D), lambda i: (i, 0)),
        ],
        out_shape=[
            jax.ShapeDtypeStruct((N_NODES, D), jnp.float32),
            jax.ShapeDtypeStruct((N_NODES, D), jnp.float32),
        ],
    )(s_embed, r_embed, W_s, b_s.reshape(1, D), W_r, b_r.reshape(1, D))


def _emm_body(e_ref, we_ref, out_ref):
    y = jnp.dot(e_ref[...], we_ref[...], preferred_element_type=jnp.float32)
    out_ref[...] = y.astype(jnp.bfloat16)


def _emm(e_embed, W_e):
    RB = 4000
    grid = (N_EDGES // RB,)
    return pl.pallas_call(
        _emm_body,
        grid=grid,
        in_specs=[
            pl.BlockSpec((RB, D), lambda i: (i, 0)),
            pl.BlockSpec((D, D), lambda i: (0, 0)),
        ],
        out_specs=pl.BlockSpec((RB, D), lambda i: (i, 0)),
        out_shape=jax.ShapeDtypeStruct((N_EDGES, D), jnp.bfloat16),
    )(e_embed, W_e)


def _hsum(x):
    for sh in (8, 4, 2, 1):
        idx = lax.iota(jnp.int32, L) ^ sh
        x = x + x[idx]
    return x


def _vrsqrt(x):
    xi = lax.bitcast_convert_type(x, jnp.int32)
    yi = jnp.full((L,), 0x5F3759DF, jnp.int32) - (xi >> 1)
    y = lax.bitcast_convert_type(yi, jnp.float32)
    half = x * 0.5
    for _ in range(2):
        y = y * (1.5 - half * y * y)
    return y


def _fused_body(ps_hbm, pr_hbm, e_hbm, snd_hbm, rcv_hbm, lns_hbm, lnb_hbm,
                out_hbm,
                sidx0, sidx1, ridx0, ridx1,
                bs0, bs1, bs2, br0, br1, be0, be1,
                lns_v, lnb_v,
                acc,
                sem_s0, sem_s1, sem_s2, sem_r0, sem_r1, sem_e0, sem_e1,
                sem_sc):
    c = lax.axis_index("c")
    s = lax.axis_index("s")
    w = s * NC + c
    ebase = w * TILE_EDGES

    sidx = (sidx0, sidx1)
    ridx = (ridx0, ridx1)
    bs = (bs0, bs1, bs2)
    br = (br0, br1)
    be = (be0, be1)
    sem_s = (sem_s0, sem_s1, sem_s2)
    sem_r = (sem_r0, sem_r1)
    sem_e = (sem_e0, sem_e1)

    def zero_row(r, carry):
        for cc in range(NCH):
            bs0[r, pl.ds(cc * L, L)] = jnp.zeros((L,), jnp.float32)
        return carry

    lax.fori_loop(0, EB, zero_row, 0)
    base_row = s * ROWS_PER_TILE
    off = 0
    for chunk in (64,) * 9 + (48,):
        pltpu.sync_copy(bs0.at[pl.ds(0, chunk)],
                        acc.at[pl.ds(base_row + off, chunk)])
        off += chunk

    @pl.when(s == NS - 1)
    def _zero_tail():
        pltpu.sync_copy(bs0.at[pl.ds(0, TAIL_ROWS)],
                        acc.at[pl.ds(NS * ROWS_PER_TILE, TAIL_ROWS)])

    pltpu.sync_copy(lns_hbm, lns_v)
    pltpu.sync_copy(lnb_hbm, lnb_v)
    lns = [lns_v[pl.ds(cc * L, L)] for cc in range(NCH)]
    lnb = [lnb_v[pl.ds(cc * L, L)] for cc in range(NCH)]

    plsc.subcore_barrier()

    def load_super(p, sp):
        base = ebase + sp * (SUP * EB)
        pltpu.sync_copy(snd_hbm.at[pl.ds(base, SUP * EB)], sidx[p])
        pltpu.sync_copy(rcv_hbm.at[pl.ds(base, SUP * EB)], ridx[p])

    def prefetch(kk, blk):
        b3, b2, p, o = kk % 3, kk % 2, (kk // SUP) % 2, (kk % SUP) * EB
        eoff = (ebase + blk * EB) * (D // 2)
        pltpu.async_copy(ps_hbm.at[sidx[p].at[pl.ds(o, EB)]], bs[b3],
                         sem_s[b3])
        pltpu.async_copy(pr_hbm.at[ridx[p].at[pl.ds(o, EB)]], br[b2],
                         sem_r[b2])
        pltpu.async_copy(e_hbm.at[pl.ds(eoff, EB * D // 2)], be[b2],
                         sem_e[b2])

    def wait_in(kk, blk):
        b3, b2, p, o = kk % 3, kk % 2, (kk // SUP) % 2, (kk % SUP) * EB
        eoff = (ebase + blk * EB) * (D // 2)
        pltpu.make_async_copy(ps_hbm.at[sidx[p].at[pl.ds(o, EB)]], bs[b3],
                              sem_s[b3]).wait()
        pltpu.make_async_copy(pr_hbm.at[ridx[p].at[pl.ds(o, EB)]], br[b2],
                              sem_r[b2]).wait()
        pltpu.make_async_copy(e_hbm.at[pl.ds(eoff, EB * D // 2)], be[b2],
                              sem_e[b2]).wait()

    def compute(kk):
        b3, b2 = kk % 3, kk % 2

        @plsc.parallel_loop(0, EB, 1, unroll=2)
        def row(r):
            v = []
            acc_s = None
            acc_q = None
            for cc in range(NCH):
                sl = pl.ds(cc * L, L)
                x = bs[b3][r, sl] + br[b2][r, sl]
                v.append(x)
                acc_s = x if cc == 0 else acc_s + x
                acc_q = x * x if cc == 0 else acc_q + x * x
            mean_v = _hsum(acc_s) * (1.0 / D)
            msq_v = _hsum(acc_q) * (1.0 / D)
            var_v = msq_v - mean_v * mean_v
            inv = _vrsqrt(var_v + 1e-6)
            for g in range(4):
                eo = pl.multiple_of(r * (D // 2) + g * L, L)
                ew = be[b2][pl.ds(eo, L)]
                e0 = lax.bitcast_convert_type(ew << 16, jnp.float32)
                e1 = lax.bitcast_convert_type(
                    ew & jnp.full((L,), -65536, jnp.int32), jnp.float32)
                for h, ev in ((0, e0), (1, e1)):
                    cc = g * 2 + h
                    sl = pl.ds(cc * L, L)
                    y = (v[cc] - mean_v) * (inv * lns[cc]) + lnb[cc]
                    y = jnp.maximum(y, 0.0)
                    bs[b3][r, sl] = y * ev

    def scatter_descs(kk):
        b3, p, o = kk % 3, (kk // SUP) % 2, (kk % SUP) * EB
        descs = []
        for q in range(4):
            idxv = ridx[p][pl.ds(o + q * L, L)]
            descs.append(
                pltpu.make_async_copy(bs[b3].at[pl.ds(q * L, L)],
                                      acc.at[idxv], sem_sc))
        return descs

    def scatter_start(kk):
        b3, p, o = kk % 3, (kk // SUP) % 2, (kk % SUP) * EB
        for q in range(4):
            idxv = ridx[p][pl.ds(o + q * L, L)]
            pltpu.async_copy(bs[b3].at[pl.ds(q * L, L)], acc.at[idxv],
                             sem_sc, add=True)

    def scatter_wait(kk):
        for d in scatter_descs(kk):
            d.wait()

    load_super(0, 0)
    prefetch(0, 0)
    prefetch(1, 1)

    def pair_body(sp2, carry):
        blk0 = sp2 * PAIR
        for kk in range(PAIR):
            if kk == 0:
                load_super(1, 2 * sp2 + 1)
            if kk == SUP:
                @pl.when(sp2 < NPAIRS - 1)
                def _load_next_a():
                    load_super(0, 2 * sp2 + 2)
            blk = blk0 + kk
            wait_in(kk, blk)
            compute(kk)
            scatter_start(kk)
            if kk == 0:
                @pl.when(sp2 > 0)
                def _drain_prev():
                    scatter_wait(5)
            else:
                scatter_wait(kk - 1)
            pf = kk + 2
            if pf < PAIR:
                prefetch(pf, blk + 2)
            else:
                @pl.when(sp2 < NPAIRS - 1)
                def _pf_next_a():
                    prefetch(pf % PAIR, blk + 2)
        return carry

    lax.fori_loop(0, NPAIRS, pair_body, 0)
    scatter_wait(5)

    @pl.when(w < LEFTOVER)
    def _tail_blocks():
        base = LEFT_BASE + w * EB
        pltpu.sync_copy(snd_hbm.at[pl.ds(base, EB)],
                        sidx0.at[pl.ds(0, EB)])
        pltpu.sync_copy(rcv_hbm.at[pl.ds(base, EB)],
                        ridx0.at[pl.ds(0, EB)])
        pltpu.async_copy(ps_hbm.at[sidx0.at[pl.ds(0, EB)]], bs0, sem_s0)
        pltpu.async_copy(pr_hbm.at[ridx0.at[pl.ds(0, EB)]], br0, sem_r0)
        pltpu.async_copy(e_hbm.at[pl.ds(base * (D // 2), EB * D // 2)], be0,
                         sem_e0)
        pltpu.make_async_copy(ps_hbm.at[sidx0.at[pl.ds(0, EB)]], bs0,
                              sem_s0).wait()
        pltpu.make_async_copy(pr_hbm.at[ridx0.at[pl.ds(0, EB)]], br0,
                              sem_r0).wait()
        pltpu.make_async_copy(e_hbm.at[pl.ds(base * (D // 2), EB * D // 2)],
                              be0, sem_e0).wait()
        compute(0)
        scatter_start(0)
        scatter_wait(0)

    plsc.subcore_barrier()

    pltpu.sync_copy(acc.at[pl.ds(base_row, ROWS_PER_TILE)],
                    out_hbm.at[c, pl.ds(base_row, ROWS_PER_TILE)])

    @pl.when(s == NS - 1)
    def _flush_tail():
        pltpu.sync_copy(acc.at[pl.ds(NS * ROWS_PER_TILE, TAIL_ROWS)],
                        out_hbm.at[c, pl.ds(NS * ROWS_PER_TILE, TAIL_ROWS)])


def _fused_sc(P_s, P_r, E, senders, receivers, ln_scale, ln_bias):
    mesh = plsc.VectorSubcoreMesh(**_SC_MESH)
    f = pl.kernel(
        _fused_body,
        out_type=jax.ShapeDtypeStruct((NC, N_NODES, D), jnp.float32),
        mesh=mesh,
        scratch_types=[
            pltpu.VMEM((SUP * EB,), jnp.int32),
            pltpu.VMEM((SUP * EB,), jnp.int32),
            pltpu.VMEM((SUP * EB,), jnp.int32),
            pltpu.VMEM((SUP * EB,), jnp.int32),
            pltpu.VMEM((EB, D), jnp.float32),
            pltpu.VMEM((EB, D), jnp.float32),
            pltpu.VMEM((EB, D), jnp.float32),
            pltpu.VMEM((EB, D), jnp.float32),
            pltpu.VMEM((EB, D), jnp.float32),
            pltpu.VMEM((EB * D // 2,), jnp.int32),
            pltpu.VMEM((EB * D // 2,), jnp.int32),
            pltpu.VMEM((D,), jnp.float32),
            pltpu.VMEM((D,), jnp.float32),
            pltpu.VMEM_SHARED((N_NODES, D), jnp.float32),
            pltpu.SemaphoreType.DMA,
            pltpu.SemaphoreType.DMA,
            pltpu.SemaphoreType.DMA,
            pltpu.SemaphoreType.DMA,
            pltpu.SemaphoreType.DMA,
            pltpu.SemaphoreType.DMA,
            pltpu.SemaphoreType.DMA,
            pltpu.SemaphoreType.DMA,
        ],
    )
    return f(P_s, P_r, E, senders, receivers, ln_scale, ln_bias)


def _final_body(p_ref, norm_ref, wo_ref, o_ref):
    m = (p_ref[0] + p_ref[1]) * norm_ref[...]
    o_ref[...] = jnp.maximum(
        jnp.dot(m, wo_ref[...], preferred_element_type=jnp.float32), 0.0)


def _finalize(partials, norm, W_out):
    RB = 2000
    grid = (N_NODES // RB,)
    return pl.pallas_call(
        _final_body,
        grid=grid,
        in_specs=[
            pl.BlockSpec((NC, RB, D), lambda i: (0, i, 0)),
            pl.BlockSpec((RB, 1), lambda i: (i, 0)),
            pl.BlockSpec((D, D), lambda i: (0, 0)),
        ],
        out_specs=pl.BlockSpec((RB, D), lambda i: (i, 0)),
        out_shape=jax.ShapeDtypeStruct((N_NODES, D), jnp.float32),
    )(partials, norm.reshape(N_NODES, 1), W_out)


def kernel(s_embed, r_embed, e_embed, norm, senders, receivers,
           W_s, b_s, W_r, b_r, ln_scale, ln_bias, W_e, W_out):
    senders = senders.astype(jnp.int32)
    receivers = receivers.astype(jnp.int32)
    P_s, P_r = _project(s_embed, r_embed, W_s, b_s, W_r, b_r)
    perm = jnp.arange(D).reshape(4, 2, L).transpose(0, 2, 1).reshape(D)
    E = lax.bitcast_convert_type(
        _emm(e_embed, W_e[:, perm]).reshape(N_EDGES * D // 2, 2), jnp.int32)
    partials = _fused_sc(P_s, P_r, E, senders, receivers, ln_scale, ln_bias)
    return _finalize(partials, norm, W_out)


# speedup vs baseline: 24.2051x; 24.2051x over previous
import jax
import jax.numpy as jnp
from jax import lax
from jax.experimental import pallas as pl
from jax.experimental.pallas import tpu as pltpu
from jax.experimental.pallas import tpu_sc as plsc

N_NODES = 10000
N_EDGES = 320000
D = 128
L = 16
NCH = D // L

NC = 2
NS = 16
NW = NC * NS
EB = 64
NBLK = N_EDGES // EB
BPT = 156
SUP = 3
PAIR = 2 * SUP
NPAIRS = BPT // PAIR
TILE_EDGES = BPT * EB
LEFTOVER = NBLK - BPT * NW
LEFT_BASE = BPT * NW * EB
ROWS_PER_TILE = 624
TAIL_ROWS = N_NODES - NS * ROWS_PER_TILE

_SC_MESH = dict(core_axis_name="c", subcore_axis_name="s",
                num_cores=NC, num_subcores=NS)


def _proj_body(s_ref, r_ref, ws_ref, bs_ref, wr_ref, br_ref, ps_ref, pr_ref):
    ps_ref[...] = jnp.dot(s_ref[...], ws_ref[...],
                          preferred_element_type=jnp.float32) + bs_ref[...]
    pr_ref[...] = jnp.dot(r_ref[...], wr_ref[...],
                          preferred_element_type=jnp.float32) + br_ref[...]


def _project(s_embed, r_embed, W_s, b_s, W_r, b_r):
    RB = 2000
    grid = (N_NODES // RB,)
    return pl.pallas_call(
        _proj_body,
        grid=grid,
        in_specs=[
            pl.BlockSpec((RB, D), lambda i: (i, 0)),
            pl.BlockSpec((RB, D), lambda i: (i, 0)),
            pl.BlockSpec((D, D), lambda i: (0, 0)),
            pl.BlockSpec((1, D), lambda i: (0, 0)),
            pl.BlockSpec((D, D), lambda i: (0, 0)),
            pl.BlockSpec((1, D), lambda i: (0, 0)),
        ],
        out_specs=[
            pl.BlockSpec((RB, D), lambda i: (i, 0)),
            pl.BlockSpec((RB, D), lambda i: (i, 0)),
        ],
        out_shape=[
            jax.ShapeDtypeStruct((N_NODES, D), jnp.float32),
            jax.ShapeDtypeStruct((N_NODES, D), jnp.float32),
        ],
    )(s_embed, r_embed, W_s, b_s.reshape(1, D), W_r, b_r.reshape(1, D))


def _emm_body(e_ref, we_ref, out_ref):
    y = jnp.dot(e_ref[...], we_ref[...], preferred_element_type=jnp.float32)
    bits = lax.bitcast_convert_type(y, jnp.int32) + 0x8000
    b3 = bits.reshape(y.shape[0] // 2, 2, D)
    out_ref[...] = ((b3[:, 1, :] & jnp.int32(-65536)) |
                    ((b3[:, 0, :] >> 16) & jnp.int32(0xFFFF)))


def _emm(e_embed, W_e):
    RB = 4000
    grid = (N_EDGES // RB,)
    return pl.pallas_call(
        _emm_body,
        grid=grid,
        in_specs=[
            pl.BlockSpec((RB, D), lambda i: (i, 0)),
            pl.BlockSpec((D, D), lambda i: (0, 0)),
        ],
        out_specs=pl.BlockSpec((RB // 2, D), lambda i: (i, 0)),
        out_shape=jax.ShapeDtypeStruct((N_EDGES // 2, D), jnp.int32),
    )(e_embed, W_e)


def _hsum(x):
    for sh in (8, 4, 2, 1):
        idx = lax.iota(jnp.int32, L) ^ sh
        x = x + x[idx]
    return x


def _vrsqrt(x):
    xi = lax.bitcast_convert_type(x, jnp.int32)
    yi = jnp.full((L,), 0x5F3759DF, jnp.int32) - (xi >> 1)
    y = lax.bitcast_convert_type(yi, jnp.float32)
    half = x * 0.5
    for _ in range(2):
        y = y * (1.5 - half * y * y)
    return y


def _fused_body(ps_hbm, pr_hbm, e_hbm, snd_hbm, rcv_hbm, lns_hbm, lnb_hbm,
                out_hbm,
                sidx0, sidx1, ridx0, ridx1,
                bs0, bs1, bs2, br0, br1, be0, be1,
                lns_v, lnb_v,
                acc,
                sem_s0, sem_s1, sem_s2, sem_r0, sem_r1, sem_e0, sem_e1,
                sem_sc):
    c = lax.axis_index("c")
    s = lax.axis_index("s")
    w = s * NC + c
    ebase = w * TILE_EDGES

    sidx = (sidx0, sidx1)
    ridx = (ridx0, ridx1)
    bs = (bs0, bs1, bs2)
    br = (br0, br1)
    be = (be0, be1)
    sem_s = (sem_s0, sem_s1, sem_s2)
    sem_r = (sem_r0, sem_r1)
    sem_e = (sem_e0, sem_e1)

    def zero_row(r, carry):
        for cc in range(NCH):
            bs0[r, pl.ds(cc * L, L)] = jnp.zeros((L,), jnp.float32)
        return carry

    lax.fori_loop(0, EB, zero_row, 0)
    base_row = s * ROWS_PER_TILE
    off = 0
    for chunk in (64,) * 9 + (48,):
        pltpu.sync_copy(bs0.at[pl.ds(0, chunk)],
                        acc.at[pl.ds(base_row + off, chunk)])
        off += chunk

    @pl.when(s == NS - 1)
    def _zero_tail():
        pltpu.sync_copy(bs0.at[pl.ds(0, TAIL_ROWS)],
                        acc.at[pl.ds(NS * ROWS_PER_TILE, TAIL_ROWS)])

    pltpu.sync_copy(lns_hbm, lns_v)
    pltpu.sync_copy(lnb_hbm, lnb_v)
    lns = [lns_v[pl.ds(cc * L, L)] for cc in range(NCH)]
    lnb = [lnb_v[pl.ds(cc * L, L)] for cc in range(NCH)]

    plsc.subcore_barrier()

    def load_super(p, sp):
        base = ebase + sp * (SUP * EB)
        pltpu.sync_copy(snd_hbm.at[pl.ds(base, SUP * EB)], sidx[p])
        pltpu.sync_copy(rcv_hbm.at[pl.ds(base, SUP * EB)], ridx[p])

    def prefetch(kk, blk):
        b3, b2, p, o = kk % 3, kk % 2, (kk // SUP) % 2, (kk % SUP) * EB
        eoff = w * (TILE_EDGES // 2) + blk * (EB // 2)
        pltpu.async_copy(ps_hbm.at[sidx[p].at[pl.ds(o, EB)]], bs[b3],
                         sem_s[b3])
        pltpu.async_copy(pr_hbm.at[ridx[p].at[pl.ds(o, EB)]], br[b2],
                         sem_r[b2])
        pltpu.async_copy(e_hbm.at[pl.ds(eoff, EB // 2)], be[b2], sem_e[b2])

    def wait_in(kk, blk):
        b3, b2, p, o = kk % 3, kk % 2, (kk // SUP) % 2, (kk % SUP) * EB
        eoff = w * (TILE_EDGES // 2) + blk * (EB // 2)
        pltpu.make_async_copy(ps_hbm.at[sidx[p].at[pl.ds(o, EB)]], bs[b3],
                              sem_s[b3]).wait()
        pltpu.make_async_copy(pr_hbm.at[ridx[p].at[pl.ds(o, EB)]], br[b2],
                              sem_r[b2]).wait()
        pltpu.make_async_copy(e_hbm.at[pl.ds(eoff, EB // 2)], be[b2],
                              sem_e[b2]).wait()

    def compute(kk):
        b3, b2 = kk % 3, kk % 2

        @plsc.parallel_loop(0, EB // 2, 1, unroll=1)
        def rowpair(t):
            for half in (0, 1):
                r = t * 2 + half
                v = []
                acc_s = None
                acc_q = None
                for cc in range(NCH):
                    sl = pl.ds(cc * L, L)
                    x = bs[b3][r, sl] + br[b2][r, sl]
                    v.append(x)
                    acc_s = x if cc == 0 else acc_s + x
                    acc_q = x * x if cc == 0 else acc_q + x * x
                mean_v = _hsum(acc_s) * (1.0 / D)
                msq_v = _hsum(acc_q) * (1.0 / D)
                var_v = msq_v - mean_v * mean_v
                inv = _vrsqrt(var_v + 1e-6)
                for cc in range(NCH):
                    sl = pl.ds(cc * L, L)
                    ew = be[b2][t, sl]
                    if half == 0:
                        ev = lax.bitcast_convert_type(ew << 16, jnp.float32)
                    else:
                        ev = lax.bitcast_convert_type(
                            ew & jnp.full((L,), -65536, jnp.int32),
                            jnp.float32)
                    y = (v[cc] - mean_v) * (inv * lns[cc]) + lnb[cc]
                    y = jnp.maximum(y, 0.0)
                    bs[b3][r, sl] = y * ev

    def scatter_descs(kk):
        b3, p, o = kk % 3, (kk // SUP) % 2, (kk % SUP) * EB
        descs = []
        for q in range(4):
            idxv = ridx[p][pl.ds(o + q * L, L)]
            descs.append(
                pltpu.make_async_copy(bs[b3].at[pl.ds(q * L, L)],
                                      acc.at[idxv], sem_sc))
        return descs

    def scatter_start(kk):
        b3, p, o = kk % 3, (kk // SUP) % 2, (kk % SUP) * EB
        for q in range(4):
            idxv = ridx[p][pl.ds(o + q * L, L)]
            pltpu.async_copy(bs[b3].at[pl.ds(q * L, L)], acc.at[idxv],
                             sem_sc, add=True)

    def scatter_wait(kk):
        for d in scatter_descs(kk):
            d.wait()

    load_super(0, 0)
    prefetch(0, 0)
    prefetch(1, 1)

    def pair_body(sp2, carry):
        blk0 = sp2 * PAIR
        for kk in range(PAIR):
            if kk == 0:
                load_super(1, 2 * sp2 + 1)
            if kk == SUP:
                @pl.when(sp2 < NPAIRS - 1)
                def _load_next_a():
                    load_super(0, 2 * sp2 + 2)
            blk = blk0 + kk
            wait_in(kk, blk)
            compute(kk)
            scatter_start(kk)
            if kk == 0:
                @pl.when(sp2 > 0)
                def _drain_prev():
                    scatter_wait(5)
            else:
                scatter_wait(kk - 1)
            pf = kk + 2
            if pf < PAIR:
                prefetch(pf, blk + 2)
            else:
                @pl.when(sp2 < NPAIRS - 1)
                def _pf_next_a():
                    prefetch(pf % PAIR, blk + 2)
        return carry

    lax.fori_loop(0, NPAIRS, pair_body, 0)
    scatter_wait(5)

    @pl.when(w < LEFTOVER)
    def _tail_blocks():
        base = LEFT_BASE + w * EB
        pltpu.sync_copy(snd_hbm.at[pl.ds(base, EB)],
                        sidx0.at[pl.ds(0, EB)])
        pltpu.sync_copy(rcv_hbm.at[pl.ds(base, EB)],
                        ridx0.at[pl.ds(0, EB)])
        pltpu.async_copy(ps_hbm.at[sidx0.at[pl.ds(0, EB)]], bs0, sem_s0)
        pltpu.async_copy(pr_hbm.at[ridx0.at[pl.ds(0, EB)]], br0, sem_r0)
        pltpu.async_copy(e_hbm.at[pl.ds(LEFT_BASE // 2 + w * (EB // 2), EB // 2)], be0, sem_e0)
        pltpu.make_async_copy(ps_hbm.at[sidx0.at[pl.ds(0, EB)]], bs0,
                              sem_s0).wait()
        pltpu.make_async_copy(pr_hbm.at[ridx0.at[pl.ds(0, EB)]], br0,
                              sem_r0).wait()
        pltpu.make_async_copy(e_hbm.at[pl.ds(LEFT_BASE // 2 + w * (EB // 2), EB // 2)], be0,
                              sem_e0).wait()
        compute(0)
        scatter_start(0)
        scatter_wait(0)

    plsc.subcore_barrier()

    pltpu.sync_copy(acc.at[pl.ds(base_row, ROWS_PER_TILE)],
                    out_hbm.at[c, pl.ds(base_row, ROWS_PER_TILE)])

    @pl.when(s == NS - 1)
    def _flush_tail():
        pltpu.sync_copy(acc.at[pl.ds(NS * ROWS_PER_TILE, TAIL_ROWS)],
                        out_hbm.at[c, pl.ds(NS * ROWS_PER_TILE, TAIL_ROWS)])


def _fused_sc(P_s, P_r, E, senders, receivers, ln_scale, ln_bias):
    mesh = plsc.VectorSubcoreMesh(**_SC_MESH)
    f = pl.kernel(
        _fused_body,
        out_type=jax.ShapeDtypeStruct((NC, N_NODES, D), jnp.float32),
        mesh=mesh,
        scratch_types=[
            pltpu.VMEM((SUP * EB,), jnp.int32),
            pltpu.VMEM((SUP * EB,), jnp.int32),
            pltpu.VMEM((SUP * EB,), jnp.int32),
            pltpu.VMEM((SUP * EB,), jnp.int32),
            pltpu.VMEM((EB, D), jnp.float32),
            pltpu.VMEM((EB, D), jnp.float32),
            pltpu.VMEM((EB, D), jnp.float32),
            pltpu.VMEM((EB, D), jnp.float32),
            pltpu.VMEM((EB, D), jnp.float32),
            pltpu.VMEM((EB // 2, D), jnp.int32),
            pltpu.VMEM((EB // 2, D), jnp.int32),
            pltpu.VMEM((D,), jnp.float32),
            pltpu.VMEM((D,), jnp.float32),
            pltpu.VMEM_SHARED((N_NODES, D), jnp.float32),
            pltpu.SemaphoreType.DMA,
            pltpu.SemaphoreType.DMA,
            pltpu.SemaphoreType.DMA,
            pltpu.SemaphoreType.DMA,
            pltpu.SemaphoreType.DMA,
            pltpu.SemaphoreType.DMA,
            pltpu.SemaphoreType.DMA,
            pltpu.SemaphoreType.DMA,
        ],
    )
    return f(P_s, P_r, E, senders, receivers, ln_scale, ln_bias)


def _final_body(p_ref, norm_ref, wo_ref, o_ref):
    m = (p_ref[0] + p_ref[1]) * norm_ref[...]
    o_ref[...] = jnp.maximum(
        jnp.dot(m, wo_ref[...], preferred_element_type=jnp.float32), 0.0)


def _finalize(partials, norm, W_out):
    RB = 2000
    grid = (N_NODES // RB,)
    return pl.pallas_call(
        _final_body,
        grid=grid,
        in_specs=[
            pl.BlockSpec((NC, RB, D), lambda i: (0, i, 0)),
            pl.BlockSpec((RB, 1), lambda i: (i, 0)),
            pl.BlockSpec((D, D), lambda i: (0, 0)),
        ],
        out_specs=pl.BlockSpec((RB, D), lambda i: (i, 0)),
        out_shape=jax.ShapeDtypeStruct((N_NODES, D), jnp.float32),
    )(partials, norm.reshape(N_NODES, 1), W_out)


def kernel(s_embed, r_embed, e_embed, norm, senders, receivers,
           W_s, b_s, W_r, b_r, ln_scale, ln_bias, W_e, W_out):
    senders = senders.astype(jnp.int32)
    receivers = receivers.astype(jnp.int32)
    P_s, P_r = _project(s_embed, r_embed, W_s, b_s, W_r, b_r)
    E = _emm(e_embed, W_e)
    partials = _fused_sc(P_s, P_r, E, senders, receivers, ln_scale, ln_bias)
    return _finalize(partials, norm, W_out)


# speedup vs baseline: 24.7422x; 1.0222x over previous
import jax
import jax.numpy as jnp
from jax import lax
from jax.experimental import pallas as pl
from jax.experimental.pallas import tpu as pltpu
from jax.experimental.pallas import tpu_sc as plsc

N_NODES = 10000
N_EDGES = 320000
D = 128
L = 16
NCH = D // L

NC = 2
NS = 16
NW = NC * NS
EB = 64
NBLK = N_EDGES // EB
BPT = 156
SUP = 3
PAIR = 2 * SUP
NPAIRS = BPT // PAIR
TILE_EDGES = BPT * EB
LEFTOVER = NBLK - BPT * NW
LEFT_BASE = BPT * NW * EB
ROWS_PER_TILE = 624
TAIL_ROWS = N_NODES - NS * ROWS_PER_TILE

_SC_MESH = dict(core_axis_name="c", subcore_axis_name="s",
                num_cores=NC, num_subcores=NS)


def _proj_body(s_ref, r_ref, ws_ref, bs_ref, wr_ref, br_ref, ps_ref, pr_ref):
    ps_ref[...] = jnp.dot(s_ref[...], ws_ref[...],
                          preferred_element_type=jnp.float32) + bs_ref[...]
    pr_ref[...] = jnp.dot(r_ref[...], wr_ref[...],
                          preferred_element_type=jnp.float32) + br_ref[...]


def _project(s_embed, r_embed, W_s, b_s, W_r, b_r):
    RB = 2000
    grid = (N_NODES // RB,)
    return pl.pallas_call(
        _proj_body,
        grid=grid,
        in_specs=[
            pl.BlockSpec((RB, D), lambda i: (i, 0)),
            pl.BlockSpec((RB, D), lambda i: (i, 0)),
            pl.BlockSpec((D, D), lambda i: (0, 0)),
            pl.BlockSpec((1, D), lambda i: (0, 0)),
            pl.BlockSpec((D, D), lambda i: (0, 0)),
            pl.BlockSpec((1, D), lambda i: (0, 0)),
        ],
        out_specs=[
            pl.BlockSpec((RB, D), lambda i: (i, 0)),
            pl.BlockSpec((RB, D), lambda i: (i, 0)),
        ],
        out_shape=[
            jax.ShapeDtypeStruct((N_NODES, D), jnp.float32),
            jax.ShapeDtypeStruct((N_NODES, D), jnp.float32),
        ],
    )(s_embed, r_embed, W_s, b_s.reshape(1, D), W_r, b_r.reshape(1, D))


def _emm_body(e_ref, we_ref, out_ref):
    out_ref[...] = jnp.dot(e_ref[...], we_ref[...],
                           preferred_element_type=jnp.float32)


def _emm(e_embed, W_e):
    RB = 4000
    grid = (N_EDGES // RB,)
    return pl.pallas_call(
        _emm_body,
        grid=grid,
        in_specs=[
            pl.BlockSpec((RB, D), lambda i: (i, 0)),
            pl.BlockSpec((D, D), lambda i: (0, 0)),
        ],
        out_specs=pl.BlockSpec((RB, D), lambda i: (i, 0)),
        out_shape=jax.ShapeDtypeStruct((N_EDGES, D), jnp.float32),
    )(e_embed, W_e)


def _hsum(x):
    for sh in (8, 4, 2, 1):
        idx = lax.iota(jnp.int32, L) ^ sh
        x = x + x[idx]
    return x


def _vrsqrt(x):
    xi = lax.bitcast_convert_type(x, jnp.int32)
    yi = jnp.full((L,), 0x5F3759DF, jnp.int32) - (xi >> 1)
    y = lax.bitcast_convert_type(yi, jnp.float32)
    half = x * 0.5
    for _ in range(2):
        y = y * (1.5 - half * y * y)
    return y


def _fused_body(ps_hbm, pr_hbm, e_hbm, snd_hbm, rcv_hbm, lns_hbm, lnb_hbm,
                out_hbm,
                sidx0, sidx1, ridx0, ridx1,
                bs0, bs1, br0, br1, be0, be1,
                lns_v, lnb_v,
                acc,
                sem_s0, sem_s1, sem_r0, sem_r1, sem_e0, sem_e1, sem_sc):
    c = lax.axis_index("c")
    s = lax.axis_index("s")
    w = s * NC + c
    ebase = w * TILE_EDGES

    sidx = (sidx0, sidx1)
    ridx = (ridx0, ridx1)
    bs = (bs0, bs1)
    br = (br0, br1)
    be = (be0, be1)
    sem_s = (sem_s0, sem_s1)
    sem_r = (sem_r0, sem_r1)
    sem_e = (sem_e0, sem_e1)

    def zero_row(r, carry):
        for cc in range(NCH):
            be0[r, pl.ds(cc * L, L)] = jnp.zeros((L,), jnp.float32)
        return carry

    lax.fori_loop(0, EB, zero_row, 0)
    base_row = s * ROWS_PER_TILE
    off = 0
    for chunk in (64,) * 9 + (48,):
        pltpu.sync_copy(be0.at[pl.ds(0, chunk)],
                        acc.at[pl.ds(base_row + off, chunk)])
        off += chunk

    @pl.when(s == NS - 1)
    def _zero_tail():
        pltpu.sync_copy(be0.at[pl.ds(0, TAIL_ROWS)],
                        acc.at[pl.ds(NS * ROWS_PER_TILE, TAIL_ROWS)])

    pltpu.sync_copy(lns_hbm, lns_v)
    pltpu.sync_copy(lnb_hbm, lnb_v)
    lns = [lns_v[pl.ds(cc * L, L)] for cc in range(NCH)]
    lnb = [lnb_v[pl.ds(cc * L, L)] for cc in range(NCH)]

    plsc.subcore_barrier()

    def load_super(p, sp):
        base = ebase + sp * (SUP * EB)
        pltpu.sync_copy(snd_hbm.at[pl.ds(base, SUP * EB)], sidx[p])
        pltpu.sync_copy(rcv_hbm.at[pl.ds(base, SUP * EB)], ridx[p])

    def prefetch(b, blk, p, local):
        o = local * EB
        eoff = ebase + blk * EB
        pltpu.async_copy(ps_hbm.at[sidx[p].at[pl.ds(o, EB)]], bs[b], sem_s[b])
        pltpu.async_copy(pr_hbm.at[ridx[p].at[pl.ds(o, EB)]], br[b], sem_r[b])
        pltpu.async_copy(e_hbm.at[pl.ds(eoff, EB)], be[b], sem_e[b])

    def wait(b, blk, p, local):
        o = local * EB
        eoff = ebase + blk * EB
        pltpu.make_async_copy(ps_hbm.at[sidx[p].at[pl.ds(o, EB)]], bs[b],
                              sem_s[b]).wait()
        pltpu.make_async_copy(pr_hbm.at[ridx[p].at[pl.ds(o, EB)]], br[b],
                              sem_r[b]).wait()
        pltpu.make_async_copy(e_hbm.at[pl.ds(eoff, EB)], be[b],
                              sem_e[b]).wait()

    def compute(b):
        @plsc.parallel_loop(0, EB, 1, unroll=2)
        def row(r):
            v = []
            acc_s = None
            acc_q = None
            for cc in range(NCH):
                sl = pl.ds(cc * L, L)
                x = bs[b][r, sl] + br[b][r, sl]
                v.append(x)
                acc_s = x if cc == 0 else acc_s + x
                acc_q = x * x if cc == 0 else acc_q + x * x
            mean_v = _hsum(acc_s) * (1.0 / D)
            msq_v = _hsum(acc_q) * (1.0 / D)
            var_v = msq_v - mean_v * mean_v
            inv = _vrsqrt(var_v + 1e-6)
            for cc in range(NCH):
                sl = pl.ds(cc * L, L)
                y = (v[cc] - mean_v) * (inv * lns[cc]) + lnb[cc]
                y = jnp.maximum(y, 0.0)
                be[b][r, sl] = y * be[b][r, sl]

    def scatter(b, p, local):
        descs = []
        for q in range(4):
            idxv = ridx[p][pl.ds(local * EB + q * L, L)]
            descs.append(
                pltpu.async_copy(be[b].at[pl.ds(q * L, L)], acc.at[idxv],
                                 sem_sc, add=True))
        for d in descs:
            d.wait()

    load_super(0, 0)
    prefetch(0, 0, 0, 0)
    prefetch(1, 1, 0, 1)

    def pair_body(sp2, carry):
        blk0 = sp2 * PAIR
        for kk in range(PAIR):
            b = kk % 2
            if kk == 0:
                load_super(1, 2 * sp2 + 1)
            if kk == SUP:
                @pl.when(sp2 < NPAIRS - 1)
                def _load_next_a():
                    load_super(0, 2 * sp2 + 2)
            blk = blk0 + kk
            wait(b, blk, kk // SUP, kk % SUP)
            compute(b)
            scatter(b, kk // SUP, kk % SUP)
            pf = kk + 2
            if pf < PAIR:
                prefetch(b, blk + 2, pf // SUP, pf % SUP)
            else:
                @pl.when(sp2 < NPAIRS - 1)
                def _pf_next_a():
                    prefetch(b, blk + 2, 0, pf % SUP)
        return carry

    lax.fori_loop(0, NPAIRS, pair_body, 0)

    @pl.when(w < LEFTOVER)
    def _tail_blocks():
        base = LEFT_BASE + w * EB
        pltpu.sync_copy(snd_hbm.at[pl.ds(base, EB)],
                        sidx0.at[pl.ds(0, EB)])
        pltpu.sync_copy(rcv_hbm.at[pl.ds(base, EB)],
                        ridx0.at[pl.ds(0, EB)])
        pltpu.async_copy(ps_hbm.at[sidx0.at[pl.ds(0, EB)]], bs0, sem_s0)
        pltpu.async_copy(pr_hbm.at[ridx0.at[pl.ds(0, EB)]], br0, sem_r0)
        pltpu.async_copy(e_hbm.at[pl.ds(base, EB)], be0, sem_e0)
        pltpu.make_async_copy(ps_hbm.at[sidx0.at[pl.ds(0, EB)]], bs0,
                              sem_s0).wait()
        pltpu.make_async_copy(pr_hbm.at[ridx0.at[pl.ds(0, EB)]], br0,
                              sem_r0).wait()
        pltpu.make_async_copy(e_hbm.at[pl.ds(base, EB)], be0, sem_e0).wait()
        compute(0)
        scatter(0, 0, 0)

    plsc.subcore_barrier()

    pltpu.sync_copy(acc.at[pl.ds(base_row, ROWS_PER_TILE)],
                    out_hbm.at[c, pl.ds(base_row, ROWS_PER_TILE)])

    @pl.when(s == NS - 1)
    def _flush_tail():
        pltpu.sync_copy(acc.at[pl.ds(NS * ROWS_PER_TILE, TAIL_ROWS)],
                        out_hbm.at[c, pl.ds(NS * ROWS_PER_TILE, TAIL_ROWS)])


def _fused_sc(P_s, P_r, E, senders, receivers, ln_scale, ln_bias):
    mesh = plsc.VectorSubcoreMesh(**_SC_MESH)
    f = pl.kernel(
        _fused_body,
        out_type=jax.ShapeDtypeStruct((NC, N_NODES, D), jnp.float32),
        mesh=mesh,
        scratch_types=[
            pltpu.VMEM((SUP * EB,), jnp.int32),
            pltpu.VMEM((SUP * EB,), jnp.int32),
            pltpu.VMEM((SUP * EB,), jnp.int32),
            pltpu.VMEM((SUP * EB,), jnp.int32),
            pltpu.VMEM((EB, D), jnp.float32),
            pltpu.VMEM((EB, D), jnp.float32),
            pltpu.VMEM((EB, D), jnp.float32),
            pltpu.VMEM((EB, D), jnp.float32),
            pltpu.VMEM((EB, D), jnp.float32),
            pltpu.VMEM((EB, D), jnp.float32),
            pltpu.VMEM((D,), jnp.float32),
            pltpu.VMEM((D,), jnp.float32),
            pltpu.VMEM_SHARED((N_NODES, D), jnp.float32),
            pltpu.SemaphoreType.DMA,
            pltpu.SemaphoreType.DMA,
            pltpu.SemaphoreType.DMA,
            pltpu.SemaphoreType.DMA,
            pltpu.SemaphoreType.DMA,
            pltpu.SemaphoreType.DMA,
            pltpu.SemaphoreType.DMA,
        ],
    )
    return f(P_s, P_r, E, senders, receivers, ln_scale, ln_bias)


def _final_body(p_ref, norm_ref, wo_ref, o_ref):
    m = (p_ref[0] + p_ref[1]) * norm_ref[...]
    o_ref[...] = jnp.maximum(
        jnp.dot(m, wo_ref[...], preferred_element_type=jnp.float32), 0.0)


def _finalize(partials, norm, W_out):
    RB = 2000
    grid = (N_NODES // RB,)
    return pl.pallas_call(
        _final_body,
        grid=grid,
        in_specs=[
            pl.BlockSpec((NC, RB, D), lambda i: (0, i, 0)),
            pl.BlockSpec((RB, 1), lambda i: (i, 0)),
            pl.BlockSpec((D, D), lambda i: (0, 0)),
        ],
        out_specs=pl.BlockSpec((RB, D), lambda i: (i, 0)),
        out_shape=jax.ShapeDtypeStruct((N_NODES, D), jnp.float32),
    )(partials, norm.reshape(N_NODES, 1), W_out)


def kernel(s_embed, r_embed, e_embed, norm, senders, receivers,
           W_s, b_s, W_r, b_r, ln_scale, ln_bias, W_e, W_out):
    senders = senders.astype(jnp.int32)
    receivers = receivers.astype(jnp.int32)
    P_s, P_r = _project(s_embed, r_embed, W_s, b_s, W_r, b_r)
    E = _emm(e_embed, W_e)
    partials = _fused_sc(P_s, P_r, E, senders, receivers, ln_scale, ln_bias)
    return _finalize(partials, norm, W_out)
